# Initial kernel scaffold; baseline (speedup 1.0000x reference)
import jax
import jax.numpy as jnp
from jax.experimental import pallas as pl


def kernel(feat, edge_index):
    raise NotImplementedError("write your pallas kernel here")


# speedup vs baseline: 4.9391x; 4.9391x over previous
import functools

import jax
import jax.numpy as jnp
from jax import lax
from jax.experimental import pallas as pl
from jax.experimental.pallas import tpu as pltpu
from jax.experimental.pallas import tpu_sc as plsc

_THETA = (1.0, -1.0, 0.5, -0.25)

_N = 10000
_E = 320000
_D = 128

_NC = 2
_NS = 16
_NW = _NC * _NS

_NP = 10240
_RPT = _NP // _NS
_CHUNK = 128
_CPT = 80
_EP = _NW * _CPT * _CHUNK

_mesh = plsc.VectorSubcoreMesh(core_axis_name="c", subcore_axis_name="s")


@functools.partial(
    pl.kernel,
    out_type=jax.ShapeDtypeStruct((_NC, _NP), jnp.float32),
    mesh=_mesh,
    scratch_types=[
        pltpu.VMEM((_CPT, _CHUNK), jnp.int32),
        pltpu.VMEM((_CHUNK,), jnp.float32),
        pltpu.VMEM((_RPT,), jnp.float32),
        pltpu.VMEM_SHARED((_NP,), jnp.float32),
        pltpu.SemaphoreType.DMA,
    ],
)
def _deg_kernel(dst_hbm, deg_hbm, dst_v, buf_v, tmp_v, deg_sh, sem):
    c = lax.axis_index("c")
    s = lax.axis_index("s")
    wid = s * _NC + c

    zero16 = jnp.zeros((16,), jnp.float32)

    def _zbuf(i, _):
        buf_v[pl.ds(i * 16, 16)] = zero16
        return 0

    lax.fori_loop(0, _CHUNK // 16, _zbuf, 0)

    def _zacc(i, _):
        pltpu.sync_copy(buf_v, deg_sh.at[pl.ds(s * _RPT + i * _CHUNK, _CHUNK)])
        return 0

    lax.fori_loop(0, _RPT // _CHUNK, _zacc, 0)

    one16 = jnp.ones((16,), jnp.float32)

    def _obuf(i, _):
        buf_v[pl.ds(i * 16, 16)] = one16
        return 0

    lax.fori_loop(0, _CHUNK // 16, _obuf, 0)

    pltpu.sync_copy(dst_hbm.at[pl.ds(wid * _CPT, _CPT)], dst_v)
    plsc.subcore_barrier()

    def _scatter(j, _):
        pltpu.sync_copy(buf_v, deg_sh.at[dst_v.at[j]], add=True)
        return 0

    lax.fori_loop(0, _CPT, _scatter, 0)
    plsc.subcore_barrier()

    pltpu.sync_copy(deg_sh.at[pl.ds(s * _RPT, _RPT)], tmp_v)
    pltpu.sync_copy(tmp_v, deg_hbm.at[c, pl.ds(s * _RPT, _RPT)])


@functools.partial(
    pl.kernel,
    out_type=jax.ShapeDtypeStruct((_NC, _NP, _D), jnp.float32),
    mesh=_mesh,
    scratch_types=[
        pltpu.VMEM((_CPT, _CHUNK), jnp.int32),
        pltpu.VMEM((_CPT, _CHUNK), jnp.int32),
        pltpu.VMEM((_CHUNK, _D), jnp.float32),
        pltpu.VMEM_SHARED((_NP, _D), jnp.float32),
        pltpu.SemaphoreType.DMA,
    ],
)
def _prop_kernel(y_hbm, src_hbm, dst_hbm, z_hbm, src_v, dst_v, rows_v, acc_sh, sem):
    c = lax.axis_index("c")
    s = lax.axis_index("s")
    wid = s * _NC + c

    zero16 = jnp.zeros((16,), jnp.float32)

    def _zrows(i, _):
        rows_v[i // 8, pl.ds((i % 8) * 16, 16)] = zero16
        return 0

    lax.fori_loop(0, _CHUNK * _D // 16, _zrows, 0)

    def _zacc(i, _):
        pltpu.sync_copy(rows_v, acc_sh.at[pl.ds(s * _RPT + i * _CHUNK, _CHUNK)])
        return 0

    lax.fori_loop(0, _RPT // _CHUNK, _zacc, 0)

    pltpu.sync_copy(src_hbm.at[pl.ds(wid * _CPT, _CPT)], src_v)
    pltpu.sync_copy(dst_hbm.at[pl.ds(wid * _CPT, _CPT)], dst_v)
    plsc.subcore_barrier()

    def _edge_chunk(j, _):
        pltpu.async_copy(y_hbm.at[src_v.at[j]], rows_v, sem).wait()
        pltpu.sync_copy(rows_v, acc_sh.at[dst_v.at[j]], add=True)
        return 0

    lax.fori_loop(0, _CPT, _edge_chunk, 0)
    plsc.subcore_barrier()

    def _wb(i, _):
        off = s * _RPT + i * _CHUNK
        pltpu.sync_copy(acc_sh.at[pl.ds(off, _CHUNK)], rows_v)
        pltpu.sync_copy(rows_v, z_hbm.at[c, pl.ds(off, _CHUNK)])
        return 0

    lax.fori_loop(0, _RPT // _CHUNK, _wb, 0)


_BLK = 640
_GRID = _NP // _BLK


def _dis_body(deg_ref, feat_ref, dis_ref, y_ref):
    d = deg_ref[...]
    tot = d[:, 0:1] + d[:, 1:2]
    dis = jnp.where(tot > 0.0, lax.rsqrt(tot), 0.0)
    disb = jnp.broadcast_to(dis, (_BLK, _D))
    dis_ref[...] = disb
    y_ref[...] = disb * feat_ref[...]


_dis_call = pl.pallas_call(
    _dis_body,
    grid=(_GRID,),
    in_specs=[
        pl.BlockSpec((_BLK, 2), lambda i: (i, 0)),
        pl.BlockSpec((_BLK, _D), lambda i: (i, 0)),
    ],
    out_specs=[
        pl.BlockSpec((_BLK, _D), lambda i: (i, 0)),
        pl.BlockSpec((_BLK, _D), lambda i: (i, 0)),
    ],
    out_shape=[
        jax.ShapeDtypeStruct((_NP, _D), jnp.float32),
        jax.ShapeDtypeStruct((_NP, _D), jnp.float32),
    ],
)


def _upd_body(theta, z_ref, dis_ref, x_ref, h_ref, xo_ref, ho_ref, yo_ref):
    zf = z_ref[0] + z_ref[1]
    dis = dis_ref[...]
    xn = x_ref[...] - dis * zf
    xo_ref[...] = xn
    ho_ref[...] = h_ref[...] + theta * xn
    yo_ref[...] = dis * xn


def _make_upd(theta):
    return pl.pallas_call(
        functools.partial(_upd_body, theta),
        grid=(_GRID,),
        in_specs=[
            pl.BlockSpec((_NC, _BLK, _D), lambda i: (0, i, 0)),
            pl.BlockSpec((_BLK, _D), lambda i: (i, 0)),
            pl.BlockSpec((_BLK, _D), lambda i: (i, 0)),
            pl.BlockSpec((_BLK, _D), lambda i: (i, 0)),
        ],
        out_specs=[
            pl.BlockSpec((_BLK, _D), lambda i: (i, 0)),
            pl.BlockSpec((_BLK, _D), lambda i: (i, 0)),
            pl.BlockSpec((_BLK, _D), lambda i: (i, 0)),
        ],
        out_shape=[
            jax.ShapeDtypeStruct((_NP, _D), jnp.float32),
            jax.ShapeDtypeStruct((_NP, _D), jnp.float32),
            jax.ShapeDtypeStruct((_NP, _D), jnp.float32),
        ],
    )


_upd_calls = tuple(_make_upd(t) for t in _THETA[1:])


def kernel(feat, edge_index):
    src = edge_index[0].astype(jnp.int32)
    dst = edge_index[1].astype(jnp.int32)

    feat_p = jnp.pad(feat, ((0, _NP - _N), (0, 0)))
    pad_idx = jnp.full((_EP - _E,), _NP - 1, jnp.int32)
    src_p = jnp.concatenate([src, pad_idx]).reshape(_EP // _CHUNK, _CHUNK)
    dst_p = jnp.concatenate([dst, pad_idx]).reshape(_EP // _CHUNK, _CHUNK)

    deg_parts = _deg_kernel(dst_p)
    dis_b, y = _dis_call(deg_parts.T, feat_p)

    x = feat_p
    h = feat_p
    for k in range(3):
        z = _prop_kernel(y, src_p, dst_p)
        x, h, y = _upd_calls[k](z, dis_b, x, h)
    return h[:_N]


# speedup vs baseline: 6.2488x; 1.2652x over previous
import functools

import jax
import jax.numpy as jnp
from jax import lax
from jax.experimental import pallas as pl
from jax.experimental.pallas import tpu as pltpu
from jax.experimental.pallas import tpu_sc as plsc

_THETA = (1.0, -1.0, 0.5, -0.25)

_N = 10000
_E = 320000
_D = 128

_NC = 2
_NS = 16
_NW = _NC * _NS

_NP = 10240
_RPT = _NP // _NS
_CHUNK = 128
_CPT = 80
_CPS = 160
_DH = _D // 2
_EP = _NW * _CPT * _CHUNK

_mesh = plsc.VectorSubcoreMesh(core_axis_name="c", subcore_axis_name="s")


@functools.partial(
    pl.kernel,
    out_type=jax.ShapeDtypeStruct((_NC, _NP), jnp.float32),
    mesh=_mesh,
    scratch_types=[
        pltpu.VMEM((_CPT, _CHUNK), jnp.int32),
        pltpu.VMEM((_CHUNK,), jnp.float32),
        pltpu.VMEM((_RPT,), jnp.float32),
        pltpu.VMEM_SHARED((_NP,), jnp.float32),
        pltpu.SemaphoreType.DMA,
    ],
)
def _deg_kernel(dst_hbm, deg_hbm, dst_v, buf_v, tmp_v, deg_sh, sem):
    c = lax.axis_index("c")
    s = lax.axis_index("s")
    wid = s * _NC + c

    zero16 = jnp.zeros((16,), jnp.float32)

    def _zbuf(i, _):
        buf_v[pl.ds(i * 16, 16)] = zero16
        return 0

    lax.fori_loop(0, _CHUNK // 16, _zbuf, 0)

    def _zacc(i, _):
        pltpu.sync_copy(buf_v, deg_sh.at[pl.ds(s * _RPT + i * _CHUNK, _CHUNK)])
        return 0

    lax.fori_loop(0, _RPT // _CHUNK, _zacc, 0)

    one16 = jnp.ones((16,), jnp.float32)

    def _obuf(i, _):
        buf_v[pl.ds(i * 16, 16)] = one16
        return 0

    lax.fori_loop(0, _CHUNK // 16, _obuf, 0)

    pltpu.sync_copy(dst_hbm.at[pl.ds(wid * _CPT, _CPT)], dst_v)
    plsc.subcore_barrier()

    def _scatter(j, _):
        pltpu.sync_copy(buf_v, deg_sh.at[dst_v.at[j]], add=True)
        return 0

    lax.fori_loop(0, _CPT, _scatter, 0)
    plsc.subcore_barrier()

    pltpu.sync_copy(deg_sh.at[pl.ds(s * _RPT, _RPT)], tmp_v)
    pltpu.sync_copy(tmp_v, deg_hbm.at[c, pl.ds(s * _RPT, _RPT)])


@functools.partial(
    pl.kernel,
    out_type=jax.ShapeDtypeStruct((_NC, _NP, _DH), jnp.float32),
    mesh=_mesh,
    scratch_types=[
        pltpu.VMEM((_CPS, _CHUNK), jnp.int32),
        pltpu.VMEM((_CPS, _CHUNK), jnp.int32),
        pltpu.VMEM((2, _CHUNK, _DH), jnp.float32),
        pltpu.VMEM_SHARED((_NP, _DH), jnp.float32),
        pltpu.SemaphoreType.DMA,
        pltpu.SemaphoreType.DMA,
    ],
    compiler_params=pltpu.CompilerParams(use_tc_tiling_on_sc=False),
)
def _prop_kernel(y_hbm, src_hbm, dst_hbm, z_hbm, src_v, dst_v, rows_v, acc_sh,
                 sem0, sem1):
    c = lax.axis_index("c")
    s = lax.axis_index("s")

    zero16 = jnp.zeros((16,), jnp.float32)

    def _zrows(i, _):
        rows_v[0, i // (_DH // 16), pl.ds((i % (_DH // 16)) * 16, 16)] = zero16
        return 0

    lax.fori_loop(0, _CHUNK * _DH // 16, _zrows, 0)

    def _zacc(i, _):
        pltpu.sync_copy(rows_v.at[0],
                        acc_sh.at[pl.ds(s * _RPT + i * _CHUNK, _CHUNK)])
        return 0

    lax.fori_loop(0, _RPT // _CHUNK, _zacc, 0)

    pltpu.sync_copy(src_hbm.at[pl.ds(s * _CPS, _CPS)], src_v)
    pltpu.sync_copy(dst_hbm.at[pl.ds(s * _CPS, _CPS)], dst_v)

    def _xform(i, _):
        j = i // (_CHUNK // 16)
        k = (i % (_CHUNK // 16)) * 16
        v = src_v[j, pl.ds(k, 16)]
        src_v[j, pl.ds(k, 16)] = v + v + c
        return 0

    lax.fori_loop(0, _CPS * _CHUNK // 16, _xform, 0)
    plsc.subcore_barrier()

    pltpu.async_copy(y_hbm.at[src_v.at[0]], rows_v.at[0], sem0)

    def _edge_pair(jj, _):
        j = 2 * jj
        pltpu.make_async_copy(y_hbm.at[src_v.at[j]], rows_v.at[0], sem0).wait()
        pltpu.async_copy(y_hbm.at[src_v.at[j + 1]], rows_v.at[1], sem1)
        pltpu.sync_copy(rows_v.at[0], acc_sh.at[dst_v.at[j]], add=True)
        pltpu.make_async_copy(
            y_hbm.at[src_v.at[j + 1]], rows_v.at[1], sem1).wait()

        @pl.when(jj + 1 < _CPS // 2)
        def _():
            pltpu.async_copy(y_hbm.at[src_v.at[j + 2]], rows_v.at[0], sem0)

        pltpu.sync_copy(rows_v.at[1], acc_sh.at[dst_v.at[j + 1]], add=True)
        return 0

    lax.fori_loop(0, _CPS // 2, _edge_pair, 0)
    plsc.subcore_barrier()

    def _wb(i, _):
        off = s * _RPT + i * _CHUNK
        pltpu.sync_copy(acc_sh.at[pl.ds(off, _CHUNK)], rows_v.at[0])
        pltpu.sync_copy(rows_v.at[0], z_hbm.at[c, pl.ds(off, _CHUNK)])
        return 0

    lax.fori_loop(0, _RPT // _CHUNK, _wb, 0)


_BLK = 640
_GRID = _NP // _BLK


def _dis_body(deg_ref, feat_ref, dis_ref, y_ref):
    d = deg_ref[...]
    tot = d[:, 0:1] + d[:, 1:2]
    dis = jnp.where(tot > 0.0, lax.rsqrt(tot), 0.0)
    disb = jnp.broadcast_to(dis, (_BLK, _D))
    dis_ref[...] = disb
    y_ref[...] = disb * feat_ref[...]


_dis_call = pl.pallas_call(
    _dis_body,
    grid=(_GRID,),
    in_specs=[
        pl.BlockSpec((_BLK, 2), lambda i: (i, 0)),
        pl.BlockSpec((_BLK, _D), lambda i: (i, 0)),
    ],
    out_specs=[
        pl.BlockSpec((_BLK, _D), lambda i: (i, 0)),
        pl.BlockSpec((_BLK, _D), lambda i: (i, 0)),
    ],
    out_shape=[
        jax.ShapeDtypeStruct((_NP, _D), jnp.float32),
        jax.ShapeDtypeStruct((_NP, _D), jnp.float32),
    ],
)


def _upd_body(theta, z_ref, dis_ref, x_ref, h_ref, xo_ref, ho_ref, yo_ref):
    zf = jnp.concatenate([z_ref[0], z_ref[1]], axis=-1)
    dis = dis_ref[...]
    xn = x_ref[...] - dis * zf
    xo_ref[...] = xn
    ho_ref[...] = h_ref[...] + theta * xn
    yo_ref[...] = dis * xn


def _make_upd(theta):
    return pl.pallas_call(
        functools.partial(_upd_body, theta),
        grid=(_GRID,),
        in_specs=[
            pl.BlockSpec((_NC, _BLK, _DH), lambda i: (0, i, 0)),
            pl.BlockSpec((_BLK, _D), lambda i: (i, 0)),
            pl.BlockSpec((_BLK, _D), lambda i: (i, 0)),
            pl.BlockSpec((_BLK, _D), lambda i: (i, 0)),
        ],
        out_specs=[
            pl.BlockSpec((_BLK, _D), lambda i: (i, 0)),
            pl.BlockSpec((_BLK, _D), lambda i: (i, 0)),
            pl.BlockSpec((_BLK, _D), lambda i: (i, 0)),
        ],
        out_shape=[
            jax.ShapeDtypeStruct((_NP, _D), jnp.float32),
            jax.ShapeDtypeStruct((_NP, _D), jnp.float32),
            jax.ShapeDtypeStruct((_NP, _D), jnp.float32),
        ],
    )


_upd_calls = tuple(_make_upd(t) for t in _THETA[1:])


def kernel(feat, edge_index):
    src = edge_index[0].astype(jnp.int32)
    dst = edge_index[1].astype(jnp.int32)

    feat_p = jnp.pad(feat, ((0, _NP - _N), (0, 0)))
    pad_idx = jnp.full((_EP - _E,), _NP - 1, jnp.int32)
    src_p = jnp.concatenate([src, pad_idx]).reshape(_EP // _CHUNK, _CHUNK)
    dst_p = jnp.concatenate([dst, pad_idx]).reshape(_EP // _CHUNK, _CHUNK)

    deg_parts = _deg_kernel(dst_p)
    dis_b, y = _dis_call(deg_parts.T, feat_p)

    x = feat_p
    h = feat_p
    for k in range(3):
        z = _prop_kernel(y.reshape(2 * _NP, _DH), src_p, dst_p)
        x, h, y = _upd_calls[k](z, dis_b, x, h)
    return h[:_N]
